# Initial kernel scaffold; baseline (speedup 1.0000x reference)
import jax
import jax.numpy as jnp
from jax.experimental import pallas as pl


def kernel(x, edge_index, edge_attr, batch, m1_W, m1_b, u1_W, u1_b, m2_W, m2_b, u2_W, u2_b, v1_W, v1_b, v2_W, v2_b):
    raise NotImplementedError("write your pallas kernel here")


# speedup vs baseline: 2.0086x; 2.0086x over previous
import functools

import jax
import jax.numpy as jnp
from jax import lax
from jax.experimental import pallas as pl
from jax.experimental.pallas import tpu as pltpu
from jax.experimental.pallas import tpu_sc as plsc

_NC = 2
_NS = 16
_NL = 16
_NW = _NC * _NS

_G = 64


def _mm_body(x_ref, w_ref, o_ref):
    o_ref[...] = jnp.dot(x_ref[...], w_ref[...],
                         preferred_element_type=jnp.float32)


def _matmul(x, w, block_rows):
    n, k = x.shape
    m = w.shape[1]
    grid = n // block_rows
    return pl.pallas_call(
        _mm_body,
        grid=(grid,),
        in_specs=[
            pl.BlockSpec((block_rows, k), lambda i: (i, 0)),
            pl.BlockSpec((k, m), lambda i: (0, 0)),
        ],
        out_specs=pl.BlockSpec((block_rows, m), lambda i: (i, 0)),
        out_shape=jax.ShapeDtypeStruct((n, m), jnp.float32),
    )(x, w)


def _em_body(e_ref, w1_ref, b1_ref, w2_ref, b2_ref, o1_ref, o2_ref):
    e = e_ref[...]
    o1_ref[...] = jnp.dot(e, w1_ref[...],
                          preferred_element_type=jnp.float32) + b1_ref[...]
    o2_ref[...] = jnp.dot(e, w2_ref[...],
                          preferred_element_type=jnp.float32) + b2_ref[...]


def _edge_proj(edge_attr, w1, b1, w2, b2, block_rows):
    e, de = edge_attr.shape
    h = w1.shape[1]
    grid = e // block_rows
    return pl.pallas_call(
        _em_body,
        grid=(grid,),
        in_specs=[
            pl.BlockSpec((block_rows, de), lambda i: (i, 0)),
            pl.BlockSpec((de, h), lambda i: (0, 0)),
            pl.BlockSpec((1, h), lambda i: (0, 0)),
            pl.BlockSpec((de, h), lambda i: (0, 0)),
            pl.BlockSpec((1, h), lambda i: (0, 0)),
        ],
        out_specs=[
            pl.BlockSpec((block_rows, h), lambda i: (i, 0)),
            pl.BlockSpec((block_rows, h), lambda i: (i, 0)),
        ],
        out_shape=[
            jax.ShapeDtypeStruct((e, h), jnp.float32),
            jax.ShapeDtypeStruct((e, h), jnp.float32),
        ],
    )(edge_attr, w1, b1, w2, b2)


def _upd_body(acc_ref, cnt_ref, x_ref, ux_ref, ua_ref, ub_ref, wn_ref,
              h_ref, xm2_ref):
    s = acc_ref[0] + acc_ref[1]
    cnt = cnt_ref[...].sum(axis=(0, 1))[:, None]
    aggr = s / jnp.maximum(cnt, 1.0)
    h = jnp.dot(x_ref[...], ux_ref[...], preferred_element_type=jnp.float32)
    h = h + jnp.dot(aggr, ua_ref[...], preferred_element_type=jnp.float32)
    h = jnp.maximum(h + ub_ref[...], 0.0)
    h_ref[...] = h
    xm2_ref[...] = jnp.dot(h, wn_ref[...], preferred_element_type=jnp.float32)


def _update(acc, cnt, x, ux, ua, ub, wnext, block_rows):
    n, h = x.shape
    grid = n // block_rows
    return pl.pallas_call(
        _upd_body,
        grid=(grid,),
        in_specs=[
            pl.BlockSpec((_NC, block_rows, h), lambda i: (0, i, 0)),
            pl.BlockSpec((_NC, _NS, block_rows), lambda i: (0, 0, i)),
            pl.BlockSpec((block_rows, h), lambda i: (i, 0)),
            pl.BlockSpec((h, h), lambda i: (0, 0)),
            pl.BlockSpec((h, h), lambda i: (0, 0)),
            pl.BlockSpec((1, h), lambda i: (0, 0)),
            pl.BlockSpec((h, h), lambda i: (0, 0)),
        ],
        out_specs=[
            pl.BlockSpec((block_rows, h), lambda i: (i, 0)),
            pl.BlockSpec((block_rows, h), lambda i: (i, 0)),
        ],
        out_shape=[
            jax.ShapeDtypeStruct((n, h), jnp.float32),
            jax.ShapeDtypeStruct((n, h), jnp.float32),
        ],
    )(acc, cnt, x, ux, ua, ub, wnext)


def _final_body(acc_ref, cnt_ref, hp_ref, b_ref, ux_ref, ua_ref, ub_ref,
                o_ref, gsum, gcnt):
    i = pl.program_id(0)
    nsteps = pl.num_programs(0)

    @pl.when(i == 0)
    def _init():
        gsum[...] = jnp.zeros_like(gsum)
        gcnt[...] = jnp.zeros_like(gcnt)

    s = acc_ref[0] + acc_ref[1]
    cnt = cnt_ref[...].sum(axis=(0, 1))[:, None]
    aggr = s / jnp.maximum(cnt, 1.0)
    h2 = jnp.dot(hp_ref[...], ux_ref[...], preferred_element_type=jnp.float32)
    h2 = h2 + jnp.dot(aggr, ua_ref[...], preferred_element_type=jnp.float32)
    h2 = jnp.maximum(h2 + ub_ref[...], 0.0)

    bb = b_ref[0, 0, :]
    nb = bb.shape[0]
    onehot = (bb[None, :] == lax.broadcasted_iota(jnp.int32, (_G, nb), 0))
    onehot = onehot.astype(jnp.float32)
    gsum[...] += jnp.dot(onehot, h2, preferred_element_type=jnp.float32,
                         precision=lax.Precision.HIGHEST)
    gcnt[...] += onehot.sum(axis=-1, keepdims=True)

    @pl.when(i == nsteps - 1)
    def _emit():
        o_ref[...] = gsum[...] / jnp.maximum(gcnt[...], 1.0)


def _final(acc, cnt, h_prev, batch3, ux, ua, ub, block_rows):
    n, h = h_prev.shape
    grid = n // block_rows
    return pl.pallas_call(
        _final_body,
        grid=(grid,),
        in_specs=[
            pl.BlockSpec((_NC, block_rows, h), lambda i: (0, i, 0)),
            pl.BlockSpec((_NC, _NS, block_rows), lambda i: (0, 0, i)),
            pl.BlockSpec((block_rows, h), lambda i: (i, 0)),
            pl.BlockSpec((1, 1, block_rows), lambda i: (i, 0, 0)),
            pl.BlockSpec((h, h), lambda i: (0, 0)),
            pl.BlockSpec((h, h), lambda i: (0, 0)),
            pl.BlockSpec((1, h), lambda i: (0, 0)),
        ],
        out_specs=pl.BlockSpec((_G, h), lambda i: (0, 0)),
        out_shape=jax.ShapeDtypeStruct((_G, h), jnp.float32),
        scratch_shapes=[
            pltpu.VMEM((_G, h), jnp.float32),
            pltpu.VMEM((_G, 1), jnp.float32),
        ],
    )(acc, cnt, h_prev, batch3, ux, ua, ub)


@functools.lru_cache(maxsize=None)
def _sc_layer(n, e, h, k, with_cnt):
    et = e // _NW
    nb = et // k
    assert et * _NW == e and nb * k == et
    hl = h // _NL
    rpt = -(-n // (_NS * 8)) * 8
    npad = rpt * _NS

    mesh = plsc.VectorSubcoreMesh(core_axis_name="c", subcore_axis_name="s")

    out_type = [jax.ShapeDtypeStruct((_NC, npad, h), jnp.float32)]
    scratch = [
        pltpu.VMEM((k,), jnp.int32),
        pltpu.VMEM((k,), jnp.int32),
        pltpu.VMEM((k, h), jnp.float32),
        pltpu.VMEM((k, h), jnp.float32),
        pltpu.VMEM_SHARED((npad, h), jnp.float32),
        pltpu.SemaphoreType.DMA,
    ]
    if with_cnt:
        out_type.append(jax.ShapeDtypeStruct((_NC, _NS, npad), jnp.float32))
        scratch.append(pltpu.VMEM((npad,), jnp.float32))

    def body(xm_h, em_h, src_h, dst_h, zh_h, *refs):
        if with_cnt:
            (acc_o, cnt_o, src_v, dst_v, msg_v, gat_v, acc_sh, sem,
             cnt_v) = refs
        else:
            acc_o, src_v, dst_v, msg_v, gat_v, acc_sh, sem = refs
        cid = lax.axis_index("c")
        sid = lax.axis_index("s")

        row0 = sid * rpt
        pltpu.sync_copy(zh_h, acc_sh.at[pl.ds(row0, rpt)])
        if with_cnt:
            zero16 = jnp.zeros((_NL,), jnp.float32)

            def zc(i, c):
                cnt_v[pl.ds(i * _NL, _NL)] = zero16
                return c

            lax.fori_loop(0, npad // _NL, zc, 0, unroll=4)
        plsc.subcore_barrier()

        tbase = (cid * _NS + sid) * et
        ones16 = jnp.ones((_NL,), jnp.float32)

        def blk(b, carry):
            base = tbase + b * k
            pltpu.sync_copy(src_h.at[pl.ds(base, k)], src_v)
            pltpu.sync_copy(dst_h.at[pl.ds(base, k)], dst_v)
            pltpu.sync_copy(em_h.at[pl.ds(base, k)], msg_v)
            pltpu.async_copy(xm_h.at[src_v], gat_v, sem).wait()

            def vec(i, c):
                r = i // hl
                col = (i % hl) * _NL
                m = msg_v[r, pl.ds(col, _NL)]
                g = gat_v[r, pl.ds(col, _NL)]
                msg_v[r, pl.ds(col, _NL)] = jnp.maximum(m + g, 0.0)
                return c

            lax.fori_loop(0, k * hl, vec, 0, unroll=4)

            pltpu.sync_copy(msg_v, acc_sh.at[dst_v], add=True)

            if with_cnt:
                def cupd(j, c):
                    idx = dst_v[pl.ds(j * _NL, _NL)]
                    plsc.addupdate_scatter(cnt_v, [idx], ones16)
                    return c

                lax.fori_loop(0, k // _NL, cupd, 0, unroll=4)
            return carry

        lax.fori_loop(0, nb, blk, 0)
        plsc.subcore_barrier()

        pltpu.sync_copy(acc_sh.at[pl.ds(row0, rpt)],
                        acc_o.at[cid, pl.ds(row0, rpt)])
        if with_cnt:
            pltpu.sync_copy(cnt_v, cnt_o.at[cid, sid])

    return pl.kernel(body, out_type=out_type, mesh=mesh,
                     scratch_types=scratch,
                     compiler_params=pltpu.CompilerParams(
                         needs_layout_passes=False))


def kernel(x, edge_index, edge_attr, batch,
           m1_W, m1_b, u1_W, u1_b,
           m2_W, m2_b, u2_W, u2_b,
           v1_W, v1_b, v2_W, v2_b):
    n, df = x.shape
    e, de = edge_attr.shape
    h = m1_W.shape[0]

    k = 80

    src = edge_index[0]
    dst = edge_index[1]

    m1x_t = m1_W[:, :df].T
    m1e_t = m1_W[:, df:].T
    m2x_t = m2_W[:, :h].T
    m2e_t = m2_W[:, h:].T
    u1x_t = u1_W[:, :df].T
    u1a_t = u1_W[:, df:].T
    u2x_t = u2_W[:, :h].T
    u2a_t = u2_W[:, h:].T
    v1_t = v1_W.T
    m1_b2 = m1_b.reshape(1, h)
    m2_b2 = m2_b.reshape(1, h)
    u1_b2 = u1_b.reshape(1, h)
    u2_b2 = u2_b.reshape(1, h)
    v1_b2 = v1_b.reshape(1, h)
    v2_row = v2_W.reshape(1, h)
    v2_b2 = v2_b.reshape(1, 1)

    blk = 1024
    npad = -(-n // blk) * blk
    rpt = npad // _NS
    zeros_h = jnp.zeros((rpt, h), jnp.float32)

    xp = jnp.pad(x, ((0, npad - n), (0, 0)))
    batch_p = jnp.pad(batch, (0, npad - n), constant_values=_G)

    xm1 = _matmul(xp, m1x_t, blk)
    em1, em2 = _edge_proj(edge_attr, m1e_t, m1_b2, m2e_t, m2_b2, 4000)

    acc1, cnt = _sc_layer(npad, e, h, k, True)(
        xm1, em1, src, dst, zeros_h)

    h1, xm2 = _update(acc1, cnt, xp, u1x_t, u1a_t, u1_b2, m2x_t, blk)

    (acc2,) = _sc_layer(npad, e, h, k, False)(
        xm2, em2, src, dst, zeros_h)

    batch3 = batch_p.reshape(npad // blk, 1, blk)
    gemb = _final(acc2, cnt, h1, batch3, u2x_t, u2a_t, u2_b2, blk)
    hid = jnp.tanh(gemb @ v1_t + v1_b2)
    return jnp.squeeze(hid @ v2_row.T + v2_b2, axis=-1)
